# baseline (speedup 1.0000x reference)
import jax
import jax.numpy as jnp
from jax.experimental import pallas as pl
from jax.experimental.pallas import tpu as pltpu

F = 161
FO = 80
W = 8
H = 16
EPS = 1e-5
TT = 192
NB = 2


def _ln_prelu(y, g, be, a):
    mu = jnp.mean(y, axis=-1, keepdims=True)
    y2 = jnp.mean(y * y, axis=-1, keepdims=True)
    var = y2 - mu * mu
    yn = (y - mu) * jax.lax.rsqrt(var + EPS)
    yn = yn * g + be
    return jnp.where(yn >= 0, yn, a * yn)


def _fused_kernel(xc_ref, xh_ref,
                  pw_w, pw_g, pw_be, pw_a,
                  w0, g0, be0, a0,
                  w1, g1, be1, a1,
                  w2, g2, be2, a2,
                  w3, g3, be3, a3,
                  we, ge, bee, ae,
                  o_ref):
    j = pl.program_id(1)
    TTp = TT + H
    xin = jnp.concatenate(
        [xh_ref[0], xc_ref[0], xh_ref[1], xc_ref[1]], axis=0)
    R = NB * TTp
    it = jax.lax.broadcasted_iota(jnp.int32, (R, 1, 1), 0)
    rmask = jnp.where((it % TTp >= H) | (j > 0), 1.0, 0.0).astype(jnp.float32)

    y = xin[:, None, :] * pw_w[...]
    src = _ln_prelu(y, pw_g[...], pw_be[...], pw_a[...]) * rmask

    cat = src.astype(jnp.bfloat16)
    out = src
    for i, (w24, g, be, a) in enumerate(((w0, g0, be0, a0), (w1, g1, be1, a1),
                                         (w2, g2, be2, a2), (w3, g3, be3, a3))):
        d = 1 << i
        Cin = cat.shape[1]
        zr = jnp.zeros((d, Cin, F), cat.dtype)
        prev = jnp.concatenate(
            [zr, cat[:TTp - d], zr, cat[TTp:2 * TTp - d]], axis=0)
        xb = jnp.concatenate([prev, cat], axis=1)
        wb = jnp.broadcast_to(w24[...], (R, 3 * W, 2 * Cin))
        z = jnp.einsum('tmk,tkf->tmf', wb, xb,
                       preferred_element_type=jnp.float32)
        zc = jnp.zeros((R, W, 1), z.dtype)
        y = (jnp.concatenate([zc, z[:, 0:W, :-1]], axis=-1)
             + z[:, W:2 * W, :]
             + jnp.concatenate([z[:, 2 * W:3 * W, 1:], zc], axis=-1))
        out = _ln_prelu(y, g[...], be[...], a[...])
        if i < 3:
            out = out * rmask
            cat = jnp.concatenate([out.astype(jnp.bfloat16), cat], axis=1)

    y3 = jnp.concatenate([out[H:TTp], out[TTp + H:]], axis=0)
    wbe = jnp.broadcast_to(we[...], (NB * TT, 3 * W, W))
    z = jnp.einsum('tmk,tkf->tmf', wbe, y3,
                   preferred_element_type=jnp.float32)
    z1c = jnp.zeros((NB * TT, W, 1), z.dtype)
    z2c = jnp.zeros((NB * TT, W, 2), z.dtype)
    zall = (z[:, 0:W, :]
            + jnp.concatenate([z[:, W:2 * W, 1:], z1c], axis=-1)
            + jnp.concatenate([z[:, 2 * W:3 * W, 2:], z2c], axis=-1))
    fr = jax.lax.broadcasted_iota(jnp.int32, (F, FO), 0)
    fc = jax.lax.broadcasted_iota(jnp.int32, (F, FO), 1)
    sel = jnp.broadcast_to((fr == 2 * fc).astype(z.dtype), (NB * TT, F, FO))
    ze = jnp.einsum('tmf,tfo->tmo', zall, sel,
                    preferred_element_type=jnp.float32)
    enc = _ln_prelu(ze, ge[...], bee[...], ae[...])
    o_ref[0] = enc[:TT]
    o_ref[1] = enc[TT:]


def kernel(x, inp_w, inp_b, inp_g, inp_be, inp_a,
           d0_w, d0_b, d0_g, d0_be, d0_a,
           d1_w, d1_b, d1_g, d1_be, d1_a,
           d2_w, d2_b, d2_g, d2_be, d2_a,
           d3_w, d3_b, d3_g, d3_be, d3_a,
           enc_w, enc_b, enc_g, enc_be, enc_a):
    B, _, T, _ = x.shape
    xs = x.reshape(B, T, F).astype(jnp.float32)

    def mk24(dw):
        ci = dw.shape[1]
        return dw.transpose(3, 0, 2, 1).reshape(3 * W, 2 * ci).astype(
            jnp.bfloat16)

    w24 = [mk24(d0_w), mk24(d1_w), mk24(d2_w), mk24(d3_w)]
    w24e = enc_w[:, :, 0, :].transpose(2, 0, 1).reshape(3 * W, W)

    def wspec(shape):
        return pl.BlockSpec(shape, lambda b, j: (0,) * len(shape))

    in_specs = [
        pl.BlockSpec((NB, TT, F), lambda b, j: (b, j, 0)),
        pl.BlockSpec((NB, H, F),
                     lambda b, j: (b, jnp.maximum(j * (TT // H) - 1, 0), 0)),
        wspec((W, 1)), wspec((1, F)), wspec((1, F)), wspec((W, 1)),
    ]
    args = [xs, xs, inp_w.reshape(W, 1), inp_g.reshape(1, F),
            inp_be.reshape(1, F), inp_a.reshape(W, 1)]
    for i, (g, be, a) in enumerate(((d0_g, d0_be, d0_a), (d1_g, d1_be, d1_a),
                                    (d2_g, d2_be, d2_a), (d3_g, d3_be, d3_a))):
        in_specs += [wspec(w24[i].shape), wspec((1, F)), wspec((1, F)),
                     wspec((W, 1))]
        args += [w24[i], g.reshape(1, F), be.reshape(1, F), a.reshape(W, 1)]
    in_specs += [wspec((3 * W, W)), wspec((1, FO)), wspec((1, FO)),
                 wspec((W, 1))]
    args += [w24e, enc_g.reshape(1, FO), enc_be.reshape(1, FO),
             enc_a.reshape(W, 1)]

    out = pl.pallas_call(
        _fused_kernel,
        out_shape=jax.ShapeDtypeStruct((B, T, W, FO), jnp.float32),
        grid=(B // NB, T // TT),
        in_specs=in_specs,
        out_specs=pl.BlockSpec((NB, TT, W, FO), lambda b, j: (b, j, 0, 0)),
        compiler_params=pltpu.CompilerParams(
            dimension_semantics=("parallel", "parallel"),
            vmem_limit_bytes=56 * 1024 * 1024),
    )(*args)
    return jnp.transpose(out, (0, 2, 1, 3))


# speedup vs baseline: 1.0832x; 1.0832x over previous
import jax
import jax.numpy as jnp
from jax.experimental import pallas as pl
from jax.experimental.pallas import tpu as pltpu

F = 161
FO = 80
W = 8
H = 16
EPS = 1e-5
TT = 256


def _ln_prelu(y, g, be, a):
    mu = jnp.mean(y, axis=-1, keepdims=True)
    y2 = jnp.mean(y * y, axis=-1, keepdims=True)
    var = y2 - mu * mu
    yn = (y - mu) * jax.lax.rsqrt(var + EPS)
    yn = yn * g + be
    return jnp.where(yn >= 0, yn, a * yn)


def _fused_kernel(xc_ref, xh_ref,
                  pw_w, pw_g, pw_be, pw_a,
                  w0, g0, be0, a0,
                  w1, g1, be1, a1,
                  w2, g2, be2, a2,
                  w3, g3, be3, a3,
                  we, ge, bee, ae,
                  o_ref):
    j = pl.program_id(1)
    xin = jnp.concatenate([xh_ref[0], xc_ref[0]], axis=0)
    TTp = xin.shape[0]
    rmask = jnp.where(
        (jax.lax.broadcasted_iota(jnp.int32, (TTp, 1, 1), 0) >= H) | (j > 0),
        1.0, 0.0).astype(jnp.float32)

    y = xin[:, None, :] * pw_w[...]
    src = _ln_prelu(y, pw_g[...], pw_be[...], pw_a[...]) * rmask

    cat = src
    out = src
    for i, (w24, g, be, a) in enumerate(((w0, g0, be0, a0), (w1, g1, be1, a1),
                                         (w2, g2, be2, a2), (w3, g3, be3, a3))):
        d = 1 << i
        Cin = cat.shape[1]
        prev = jnp.concatenate(
            [jnp.zeros((d, Cin, F), cat.dtype), cat[:TTp - d]], axis=0)
        xb = jnp.concatenate([prev, cat], axis=1)
        wb = jnp.broadcast_to(w24[...], (TTp, 3 * W, 2 * Cin))
        z = jnp.einsum('tmk,tkf->tmf', wb, xb,
                       preferred_element_type=jnp.float32)
        zc = jnp.zeros((TTp, W, 1), z.dtype)
        y = (jnp.concatenate([zc, z[:, 0:W, :-1]], axis=-1)
             + z[:, W:2 * W, :]
             + jnp.concatenate([z[:, 2 * W:3 * W, 1:], zc], axis=-1))
        out = _ln_prelu(y, g[...], be[...], a[...])
        if i < 3:
            out = out * rmask
            cat = jnp.concatenate([out, cat], axis=1)

    y3 = out[H:, :, :]
    wbe = jnp.broadcast_to(we[...], (TT, 3 * W, W))
    z = jnp.einsum('tmk,tkf->tmf', wbe, y3,
                   preferred_element_type=jnp.float32)
    z1c = jnp.zeros((TT, W, 1), z.dtype)
    z2c = jnp.zeros((TT, W, 2), z.dtype)
    zall = (z[:, 0:W, :]
            + jnp.concatenate([z[:, W:2 * W, 1:], z1c], axis=-1)
            + jnp.concatenate([z[:, 2 * W:3 * W, 2:], z2c], axis=-1))
    fr = jax.lax.broadcasted_iota(jnp.int32, (F, FO), 0)
    fc = jax.lax.broadcasted_iota(jnp.int32, (F, FO), 1)
    sel = jnp.broadcast_to((fr == 2 * fc).astype(z.dtype), (TT, F, FO))
    ze = jnp.einsum('tmf,tfo->tmo', zall, sel,
                    preferred_element_type=jnp.float32)
    o_ref[0] = _ln_prelu(ze, ge[...], bee[...], ae[...])


def kernel(x, inp_w, inp_b, inp_g, inp_be, inp_a,
           d0_w, d0_b, d0_g, d0_be, d0_a,
           d1_w, d1_b, d1_g, d1_be, d1_a,
           d2_w, d2_b, d2_g, d2_be, d2_a,
           d3_w, d3_b, d3_g, d3_be, d3_a,
           enc_w, enc_b, enc_g, enc_be, enc_a):
    B, _, T, _ = x.shape
    xs = x.reshape(B, T, F).astype(jnp.float32)

    def mk24(dw):
        ci = dw.shape[1]
        return dw.transpose(3, 0, 2, 1).reshape(3 * W, 2 * ci)

    w24 = [mk24(d0_w), mk24(d1_w), mk24(d2_w), mk24(d3_w)]
    w24e = enc_w[:, :, 0, :].transpose(2, 0, 1).reshape(3 * W, W)

    def wspec(shape):
        return pl.BlockSpec(shape, lambda b, j: (0,) * len(shape))

    in_specs = [
        pl.BlockSpec((1, TT, F), lambda b, j: (b, j, 0)),
        pl.BlockSpec((1, H, F),
                     lambda b, j: (b, jnp.maximum(j * (TT // H) - 1, 0), 0)),
        wspec((W, 1)), wspec((1, F)), wspec((1, F)), wspec((W, 1)),
    ]
    args = [xs, xs, inp_w.reshape(W, 1), inp_g.reshape(1, F),
            inp_be.reshape(1, F), inp_a.reshape(W, 1)]
    for i, (g, be, a) in enumerate(((d0_g, d0_be, d0_a), (d1_g, d1_be, d1_a),
                                    (d2_g, d2_be, d2_a), (d3_g, d3_be, d3_a))):
        in_specs += [wspec(w24[i].shape), wspec((1, F)), wspec((1, F)),
                     wspec((W, 1))]
        args += [w24[i], g.reshape(1, F), be.reshape(1, F), a.reshape(W, 1)]
    in_specs += [wspec((3 * W, W)), wspec((1, FO)), wspec((1, FO)),
                 wspec((W, 1))]
    args += [w24e, enc_g.reshape(1, FO), enc_be.reshape(1, FO),
             enc_a.reshape(W, 1)]

    out = pl.pallas_call(
        _fused_kernel,
        out_shape=jax.ShapeDtypeStruct((B, T, W, FO), jnp.float32),
        grid=(B, T // TT),
        in_specs=in_specs,
        out_specs=pl.BlockSpec((1, TT, W, FO), lambda b, j: (b, j, 0, 0)),
        compiler_params=pltpu.CompilerParams(
            dimension_semantics=("parallel", "parallel"),
            vmem_limit_bytes=56 * 1024 * 1024),
    )(*args)
    return jnp.transpose(out, (0, 2, 1, 3))


# speedup vs baseline: 1.1162x; 1.0304x over previous
import jax
import jax.numpy as jnp
from jax.experimental import pallas as pl
from jax.experimental.pallas import tpu as pltpu

F = 161
FO = 80
W = 8
H = 16
EPS = 1e-5
TT = 384


def _ln_prelu(y, g, be, a):
    mu = jnp.mean(y, axis=-1, keepdims=True)
    y2 = jnp.mean(y * y, axis=-1, keepdims=True)
    var = y2 - mu * mu
    yn = (y - mu) * jax.lax.rsqrt(var + EPS)
    yn = yn * g + be
    return jnp.where(yn >= 0, yn, a * yn)


def _fused_kernel(xc_ref, xh_ref,
                  pw_w, pw_g, pw_be, pw_a,
                  w0, g0, be0, a0,
                  w1, g1, be1, a1,
                  w2, g2, be2, a2,
                  w3, g3, be3, a3,
                  we, ge, bee, ae,
                  o_ref):
    j = pl.program_id(1)
    xin = jnp.concatenate([xh_ref[0], xc_ref[0]], axis=0)
    TTp = xin.shape[0]
    rmask = jnp.where(
        (jax.lax.broadcasted_iota(jnp.int32, (TTp, 1, 1), 0) >= H) | (j > 0),
        1.0, 0.0).astype(jnp.float32)

    y = xin[:, None, :] * pw_w[...]
    src = _ln_prelu(y, pw_g[...], pw_be[...], pw_a[...]) * rmask

    cat = src
    out = src
    for i, (w24, g, be, a) in enumerate(((w0, g0, be0, a0), (w1, g1, be1, a1),
                                         (w2, g2, be2, a2), (w3, g3, be3, a3))):
        d = 1 << i
        Cin = cat.shape[1]
        prev = jnp.concatenate(
            [jnp.zeros((d, Cin, F), cat.dtype), cat[:TTp - d]], axis=0)
        xb = jnp.concatenate([prev, cat], axis=1)
        wb = jnp.broadcast_to(w24[...], (TTp, 3 * W, 2 * Cin))
        z = jnp.einsum('tmk,tkf->tmf', wb, xb,
                       preferred_element_type=jnp.float32)
        zc = jnp.zeros((TTp, W, 1), z.dtype)
        y = (jnp.concatenate([zc, z[:, 0:W, :-1]], axis=-1)
             + z[:, W:2 * W, :]
             + jnp.concatenate([z[:, 2 * W:3 * W, 1:], zc], axis=-1))
        out = _ln_prelu(y, g[...], be[...], a[...])
        if i < 3:
            out = out * rmask
            cat = jnp.concatenate([out, cat], axis=1)

    y3 = out[H:, :, :]
    wbe = jnp.broadcast_to(we[...], (TT, 3 * W, W))
    z = jnp.einsum('tmk,tkf->tmf', wbe, y3,
                   preferred_element_type=jnp.float32)
    z1c = jnp.zeros((TT, W, 1), z.dtype)
    z2c = jnp.zeros((TT, W, 2), z.dtype)
    zall = (z[:, 0:W, :]
            + jnp.concatenate([z[:, W:2 * W, 1:], z1c], axis=-1)
            + jnp.concatenate([z[:, 2 * W:3 * W, 2:], z2c], axis=-1))
    fr = jax.lax.broadcasted_iota(jnp.int32, (F, FO), 0)
    fc = jax.lax.broadcasted_iota(jnp.int32, (F, FO), 1)
    sel = jnp.broadcast_to((fr == 2 * fc).astype(z.dtype), (TT, F, FO))
    ze = jnp.einsum('tmf,tfo->tmo', zall, sel,
                    preferred_element_type=jnp.float32)
    o_ref[0] = _ln_prelu(ze, ge[...], bee[...], ae[...])


def kernel(x, inp_w, inp_b, inp_g, inp_be, inp_a,
           d0_w, d0_b, d0_g, d0_be, d0_a,
           d1_w, d1_b, d1_g, d1_be, d1_a,
           d2_w, d2_b, d2_g, d2_be, d2_a,
           d3_w, d3_b, d3_g, d3_be, d3_a,
           enc_w, enc_b, enc_g, enc_be, enc_a):
    B, _, T, _ = x.shape
    xs = x.reshape(B, T, F).astype(jnp.float32)

    def mk24(dw):
        ci = dw.shape[1]
        return dw.transpose(3, 0, 2, 1).reshape(3 * W, 2 * ci)

    w24 = [mk24(d0_w), mk24(d1_w), mk24(d2_w), mk24(d3_w)]
    w24e = enc_w[:, :, 0, :].transpose(2, 0, 1).reshape(3 * W, W)

    def wspec(shape):
        return pl.BlockSpec(shape, lambda b, j: (0,) * len(shape))

    in_specs = [
        pl.BlockSpec((1, TT, F), lambda b, j: (b, j, 0)),
        pl.BlockSpec((1, H, F),
                     lambda b, j: (b, jnp.maximum(j * (TT // H) - 1, 0), 0)),
        wspec((W, 1)), wspec((1, F)), wspec((1, F)), wspec((W, 1)),
    ]
    args = [xs, xs, inp_w.reshape(W, 1), inp_g.reshape(1, F),
            inp_be.reshape(1, F), inp_a.reshape(W, 1)]
    for i, (g, be, a) in enumerate(((d0_g, d0_be, d0_a), (d1_g, d1_be, d1_a),
                                    (d2_g, d2_be, d2_a), (d3_g, d3_be, d3_a))):
        in_specs += [wspec(w24[i].shape), wspec((1, F)), wspec((1, F)),
                     wspec((W, 1))]
        args += [w24[i], g.reshape(1, F), be.reshape(1, F), a.reshape(W, 1)]
    in_specs += [wspec((3 * W, W)), wspec((1, FO)), wspec((1, FO)),
                 wspec((W, 1))]
    args += [w24e, enc_g.reshape(1, FO), enc_be.reshape(1, FO),
             enc_a.reshape(W, 1)]

    out = pl.pallas_call(
        _fused_kernel,
        out_shape=jax.ShapeDtypeStruct((B, T, W, FO), jnp.float32),
        grid=(B, T // TT),
        in_specs=in_specs,
        out_specs=pl.BlockSpec((1, TT, W, FO), lambda b, j: (b, j, 0, 0)),
        compiler_params=pltpu.CompilerParams(
            dimension_semantics=("parallel", "parallel"),
            vmem_limit_bytes=56 * 1024 * 1024),
    )(*args)
    return jnp.transpose(out, (0, 2, 1, 3))
